# baseline (speedup 1.0000x reference)
import jax
import jax.numpy as jnp
from jax import lax
from jax.experimental import pallas as pl
from jax.experimental.pallas import tpu as pltpu
from jax.experimental.pallas import tpu_sc as plsc

BATCH = 4096
VEC = 221
OUT = 1153
L = 16
NW = 32
RPW = BATCH // NW
NG = RPW // L


def _body(in_hbm, ce_hbm, ie_hbm, out_hbm,
          ce_v, ie_v, in_v, out_v0, out_v1, sem0, sem1):
    wid = lax.axis_index("s") * 2 + lax.axis_index("c")
    row0 = wid * RPW

    pltpu.sync_copy(ce_hbm, ce_v)
    pltpu.sync_copy(ie_hbm, ie_v)
    pltpu.sync_copy(in_hbm.at[pl.ds(row0, RPW)], in_v)

    lanes = lax.iota(jnp.int32, L)
    onef = jnp.full((L,), 1.0, jnp.float32)
    zerof = jnp.zeros((L,), jnp.float32)

    def c16(v):
        return jnp.full((L,), v, jnp.int32)

    def zboth(i, c):
        col = c16(0) + i
        plsc.store_scatter(out_v0, [lanes, col], zerof)
        plsc.store_scatter(out_v1, [lanes, col], zerof)
        return c

    lax.fori_loop(0, OUT, zboth, 0)

    def compute(g, out_v):
        rows = g * L + lanes

        def gi(col):
            return plsc.load_gather(in_v, [rows, col])

        def sst(col, val):
            plsc.store_scatter(out_v, [lanes, col], val)

        def sadd(col, val):
            plsc.addupdate_scatter(out_v, [lanes, col], val)

        posi = gi(c16(0)).astype(jnp.int32)
        tchi = gi(posi + 1).astype(jnp.int32)
        ochi = gi(posi + 6).astype(jnp.int32)

        sadd(posi, onef)
        sadd(5 + tchi, onef)
        sadd(411 + ochi, onef)
        for c in range(5):
            oc = gi(c16(6 + c)).astype(jnp.int32)
            sadd(947 + oc, onef)

        cbt = 11 + 12 * posi
        cbo = cbt + 60
        for j in range(6):
            tid = gi(cbt + 2 * j).astype(jnp.int32)
            tcnt = gi(cbt + (2 * j + 1))
            sadd(161 + tid, tcnt)
            oid = gi(cbo + 2 * j).astype(jnp.int32)
            ocnt = gi(cbo + (2 * j + 1))
            sadd(567 + oid, ocnt)

        sst(c16(1097), gi(posi + 211))
        sst(c16(1098), gi(posi + 141) + gi(posi + 151))
        kb = 181 + 3 * posi
        for k in range(3):
            sst(c16(1099 + k), gi(kb + k))
        sst(c16(1102), gi(posi + 171))

        def emb_d(d, c):
            dd = c16(0) + d
            sst(155 + dd, plsc.load_gather(ce_v, [tchi, dd]))
            sst(561 + dd, plsc.load_gather(ce_v, [ochi, dd]))
            return c

        lax.fori_loop(0, 6, emb_d, 0)

        def champ(c, cc):
            ci = gi(c16(1) + c).astype(jnp.int32)

            def ce_d(d, cc2):
                dd = c16(0) + d
                sst(817 + 6 * c + dd, plsc.load_gather(ce_v, [ci, dd]))
                return cc2

            lax.fori_loop(0, 6, ce_d, 0)
            cbc = c16(11) + 12 * c
            acc = [zerof] * 7
            for j in range(6):
                iid = gi(cbc + 2 * j).astype(jnp.int32)
                icnt = gi(cbc + (2 * j + 1))
                for d in range(7):
                    acc[d] = acc[d] + icnt * plsc.load_gather(ie_v, [iid, c16(d)])
            for d in range(7):
                sst(877 + 7 * c + c16(d), acc[d])
            return cc

        lax.fori_loop(0, 10, champ, 0)

        def dense40(w, c):
            ww = c16(0) + w
            sst(1103 + ww, gi(171 + ww))
            return c

        def dense10(w, c):
            ww = c16(0) + w
            sst(1143 + ww, gi(141 + ww))
            return c

        lax.fori_loop(0, 40, dense40, 0)
        lax.fori_loop(0, 10, dense10, 0)

    def unscatter(g, out_v):
        rows = g * L + lanes

        def gi(col):
            return plsc.load_gather(in_v, [rows, col])

        def szero(col):
            plsc.store_scatter(out_v, [lanes, col], zerof)

        posi = gi(c16(0)).astype(jnp.int32)
        tchi = gi(posi + 1).astype(jnp.int32)
        ochi = gi(posi + 6).astype(jnp.int32)
        szero(posi)
        szero(5 + tchi)
        szero(411 + ochi)
        for c in range(5):
            oc = gi(c16(6 + c)).astype(jnp.int32)
            szero(947 + oc)
        cbt = 11 + 12 * posi
        cbo = cbt + 60
        for j in range(6):
            tid = gi(cbt + 2 * j).astype(jnp.int32)
            szero(161 + tid)
            oid = gi(cbo + 2 * j).astype(jnp.int32)
            szero(567 + oid)

    def start_out(g, out_v, sem):
        base = row0 + g * L
        pltpu.async_copy(out_v, out_hbm.at[pl.ds(base, L)], sem)

    def wait_out(out_v, sem):
        pltpu.make_async_copy(
            out_v, out_hbm.at[pl.ds(0, L)], sem).wait()

    compute(0, out_v0)
    start_out(0, out_v0, sem0)
    compute(1, out_v1)
    start_out(1, out_v1, sem1)

    def pair(k, c):
        g0 = 2 * k
        wait_out(out_v0, sem0)
        unscatter(g0 - 2, out_v0)
        compute(g0, out_v0)
        start_out(g0, out_v0, sem0)
        g1 = 2 * k + 1
        wait_out(out_v1, sem1)
        unscatter(g1 - 2, out_v1)
        compute(g1, out_v1)
        start_out(g1, out_v1, sem1)
        return c

    lax.fori_loop(1, NG // 2, pair, 0)
    wait_out(out_v0, sem0)
    wait_out(out_v1, sem1)


def _make_sc_call(interpret=False):
    return pl.kernel(
        _body,
        out_type=jax.ShapeDtypeStruct((BATCH, OUT), jnp.float32),
        mesh=plsc.VectorSubcoreMesh(core_axis_name="c", subcore_axis_name="s"),
        scratch_types=[
            pltpu.VMEM((150, 6), jnp.float32),
            pltpu.VMEM((250, 7), jnp.float32),
            pltpu.VMEM((RPW, VEC), jnp.float32),
            pltpu.VMEM((L, OUT), jnp.float32),
            pltpu.VMEM((L, OUT), jnp.float32),
            pltpu.SemaphoreType.DMA,
            pltpu.SemaphoreType.DMA,
        ],
        compiler_params=pltpu.CompilerParams(
            needs_layout_passes=False,
            use_tc_tiling_on_sc=True,
        ),
        interpret=interpret,
    )


@jax.jit
def kernel(in_vec, champ_embs, item_embs):
    return _make_sc_call()(in_vec, champ_embs, item_embs)


# speedup vs baseline: 1.1195x; 1.1195x over previous
import jax
import jax.numpy as jnp
from jax import lax
from jax.experimental import pallas as pl
from jax.experimental.pallas import tpu as pltpu
from jax.experimental.pallas import tpu_sc as plsc

BATCH = 4096
VEC = 221
OUT = 1153
L = 16
NW = 32
RPW = BATCH // NW
OUTP = 1153
OUT_BLK = 18560
NG = RPW // L

CE_STRIDE = 7
CE_PAD = 1056
IE_PAD = 1760


def _body(in_hbm, ce_hbm, ie_hbm, out_hbm,
          ce_v, ie_v, in_v, out_v0, out_v1, sem0, sem1, semi):
    wid = lax.axis_index("s") * 2 + lax.axis_index("c")
    row0 = wid * RPW

    pltpu.sync_copy(ce_hbm, ce_v)
    pltpu.sync_copy(ie_hbm, ie_v)
    pltpu.sync_copy(in_hbm.at[pl.ds(row0 * VEC, RPW * VEC)], in_v)

    lanes = lax.iota(jnp.int32, L)
    ob = lanes * OUTP
    onef = jnp.full((L,), 1.0, jnp.float32)
    zerof = jnp.zeros((L,), jnp.float32)

    def zboth(i, c):
        out_v0[pl.ds(i * L, L)] = zerof
        out_v1[pl.ds(i * L, L)] = zerof
        return c

    lax.fori_loop(0, L * OUTP // L, zboth, 0)

    def compute(g, out_v):
        rb = (g * L + lanes) * VEC

        def gi(idx):
            return plsc.load_gather(in_v, [idx])

        def sst(idx, val):
            plsc.store_scatter(out_v, [ob + idx], val)

        def sadd(idx, val):
            plsc.addupdate_scatter(out_v, [ob + idx], val)

        posi = gi(rb).astype(jnp.int32)
        rbp = rb + posi
        tchi = gi(rbp + 1).astype(jnp.int32)
        ochi = gi(rbp + 6).astype(jnp.int32)

        sadd(posi, onef)
        sadd(5 + tchi, onef)
        sadd(411 + ochi, onef)
        for c in range(5):
            oc = gi(rb + (6 + c)).astype(jnp.int32)
            sadd(947 + oc, onef)

        rbt = rb + 11 + 12 * posi
        rbo = rbt + 60
        for j in range(6):
            tid = gi(rbt + 2 * j).astype(jnp.int32)
            tcnt = gi(rbt + (2 * j + 1))
            sadd(161 + tid, tcnt)
            oid = gi(rbo + 2 * j).astype(jnp.int32)
            ocnt = gi(rbo + (2 * j + 1))
            sadd(567 + oid, ocnt)

        sst(1097, gi(rbp + 211))
        sst(1098, gi(rbp + 141) + gi(rbp + 151))
        kb = rb + 181 + 3 * posi
        for k in range(3):
            sst(1099 + k, gi(kb + k))
        sst(1102, gi(rbp + 171))

        tce = tchi * CE_STRIDE
        oce = ochi * CE_STRIDE
        for d in range(6):
            sst(155 + d, plsc.load_gather(ce_v, [tce + d]))
            sst(561 + d, plsc.load_gather(ce_v, [oce + d]))

        def champ(c, cc):
            ci = gi(rb + (1 + c)).astype(jnp.int32) * CE_STRIDE
            for d in range(6):
                sst(817 + 6 * c + d, plsc.load_gather(ce_v, [ci + d]))
            rbc = rb + (11 + 12 * c)
            acc = [zerof] * 7
            for j in range(6):
                iid = gi(rbc + 2 * j).astype(jnp.int32) * 7
                icnt = gi(rbc + (2 * j + 1))
                for d in range(7):
                    acc[d] = acc[d] + icnt * plsc.load_gather(ie_v, [iid + d])
            for d in range(7):
                sst(877 + 7 * c + d, acc[d])
            return cc

        lax.fori_loop(0, 10, champ, 0)

        for w in range(40):
            sst(1103 + w, gi(rb + (171 + w)))
        for w in range(10):
            sst(1143 + w, gi(rb + (141 + w)))

    def unscatter(g, out_v):
        rb = (g * L + lanes) * VEC

        def gi(idx):
            return plsc.load_gather(in_v, [idx])

        def szero(idx):
            plsc.store_scatter(out_v, [ob + idx], zerof)

        posi = gi(rb).astype(jnp.int32)
        rbp = rb + posi
        tchi = gi(rbp + 1).astype(jnp.int32)
        ochi = gi(rbp + 6).astype(jnp.int32)
        szero(posi)
        szero(5 + tchi)
        szero(411 + ochi)
        for c in range(5):
            oc = gi(rb + (6 + c)).astype(jnp.int32)
            szero(947 + oc)
        rbt = rb + 11 + 12 * posi
        rbo = rbt + 60
        for j in range(6):
            tid = gi(rbt + 2 * j).astype(jnp.int32)
            szero(161 + tid)
            oid = gi(rbo + 2 * j).astype(jnp.int32)
            szero(567 + oid)

    def start_out(g, out_v, sem):
        base = (wid * NG + g) * OUT_BLK
        pltpu.async_copy(out_v, out_hbm.at[pl.ds(base, L * OUT)], sem)

    def wait_out(out_v, sem):
        pltpu.make_async_copy(out_v, out_hbm.at[pl.ds(0, L * OUT)],
                              sem).wait()

    compute(0, out_v0)
    start_out(0, out_v0, sem0)
    compute(1, out_v1)
    start_out(1, out_v1, sem1)

    def pair(k, c):
        g0 = 2 * k
        wait_out(out_v0, sem0)
        unscatter(g0 - 2, out_v0)
        compute(g0, out_v0)
        start_out(g0, out_v0, sem0)
        g1 = 2 * k + 1
        wait_out(out_v1, sem1)
        unscatter(g1 - 2, out_v1)
        compute(g1, out_v1)
        start_out(g1, out_v1, sem1)
        return c

    lax.fori_loop(1, NG // 2, pair, 0)
    wait_out(out_v0, sem0)
    wait_out(out_v1, sem1)


def _make_sc_call(interpret=False):
    return pl.kernel(
        _body,
        out_type=jax.ShapeDtypeStruct((BATCH // L * OUT_BLK,), jnp.float32),
        mesh=plsc.VectorSubcoreMesh(core_axis_name="c", subcore_axis_name="s"),
        scratch_types=[
            pltpu.VMEM((CE_PAD,), jnp.float32),
            pltpu.VMEM((IE_PAD,), jnp.float32),
            pltpu.VMEM((RPW * VEC,), jnp.float32),
            pltpu.VMEM((L * OUTP,), jnp.float32),
            pltpu.VMEM((L * OUTP,), jnp.float32),
            pltpu.SemaphoreType.DMA,
            pltpu.SemaphoreType.DMA,
            pltpu.SemaphoreType.DMA,
        ],
        compiler_params=pltpu.CompilerParams(
            needs_layout_passes=False,
            use_tc_tiling_on_sc=True,
        ),
        interpret=interpret,
    )


@jax.jit
def kernel(in_vec, champ_embs, item_embs):
    ce = jnp.pad(champ_embs, ((0, 0), (0, CE_STRIDE - 6))).reshape(-1)
    ce = jnp.pad(ce, (0, CE_PAD - 150 * CE_STRIDE))
    ie = jnp.pad(item_embs.reshape(-1), (0, IE_PAD - 1750))
    out = _make_sc_call()(in_vec.reshape(-1), ce, ie)
    out = out.reshape(BATCH // L, OUT_BLK)[:, :L * OUT]
    return out.reshape(BATCH, OUT)
